# baseline (speedup 1.0000x reference)
import jax
import jax.numpy as jnp
from jax import lax
from jax.experimental import pallas as pl
from jax.experimental.pallas import tpu as pltpu
from jax.experimental.pallas import tpu_sc as plsc

IN_F_STATIC = 50
CHUNK = 8
ROWS_PER_W = 5


def _vsqrt(s):
    y = 0.5 * (s + 1.0)
    for _ in range(16):
        y = 0.5 * (y + s / y)
    return y


def _lane_sum(v):
    idx = lax.broadcasted_iota(jnp.int32, (16,), 0)
    dnums = lax.GatherDimensionNumbers(
        offset_dims=(), collapsed_slice_dims=(0,), start_index_map=(0,))
    for sh in (8, 4, 2, 1):
        shuf = lax.gather(v, (idx ^ sh)[:, None], dnums, (1,),
                          mode=lax.GatherScatterMode.PROMISE_IN_BOUNDS)
        v = v + shuf
    return v


def _sc_body(table_ref, idx_ref, out_ref, idx_v, rows_v, sem):
    wid = lax.axis_index("s") * 2 + lax.axis_index("c")
    pltpu.sync_copy(idx_ref.at[wid], idx_v)
    pltpu.async_copy(table_ref.at[idx_v], rows_v, sem).wait()
    for j in range(ROWS_PER_W):
        acc = jnp.zeros((16,), jnp.float32)
        for k in range(8):
            v = rows_v[j, pl.ds(k * 16, 16)]
            acc = acc + v * v
        s = _lane_sum(acc)
        norm = _vsqrt(s)
        scale = jnp.where(norm > 1.0, 1.0 / (norm + 1e-7),
                          jnp.ones((16,), jnp.float32))
        for k in range(8):
            rows_v[j, pl.ds(k * 16, 16)] = rows_v[j, pl.ds(k * 16, 16)] * scale
    pltpu.sync_copy(rows_v, out_ref.at[wid])


def _sc_lookup_renorm(table, idx2d, n_rows):
    run = pl.kernel(
        _sc_body,
        out_type=jax.ShapeDtypeStruct(
            (n_rows // ROWS_PER_W, idx2d.shape[1], table.shape[1]), table.dtype),
        mesh=plsc.VectorSubcoreMesh(core_axis_name="c", subcore_axis_name="s"),
        scratch_types=[
            pltpu.VMEM((idx2d.shape[1],), jnp.int32),
            pltpu.VMEM((idx2d.shape[1], table.shape[1]), jnp.float32),
            pltpu.SemaphoreType.DMA,
        ],
    )
    return run(table, idx2d)


def _tc_body(x_ref, temb_ref, pemb_ref, o_ref, bias_ref):
    T, P, C = x_ref.shape[1], x_ref.shape[2], x_ref.shape[3]
    H = C // 2

    @pl.when(pl.program_id(0) == 0)
    def _build_bias():
        hr = lax.broadcasted_iota(jnp.int32, (H, C), 0)
        hc = lax.broadcasted_iota(jnp.int32, (H, C), 1)
        even = (hc == 2 * hr).astype(jnp.float32)
        odd = (hc == 2 * hr + 1).astype(jnp.float32)
        time_part = lax.dot(temb_ref[...], even, precision=lax.Precision.HIGHEST)
        pers_part = lax.dot(pemb_ref[...], odd, precision=lax.Precision.HIGHEST)
        bias_ref[...] = time_part[:, None, :] + pers_part[None, :, :]

    o_ref[...] = x_ref[...] + bias_ref[...]


def kernel(x, W_obs, W_pred, W_person, in_F, out_F, num_people):
    B, T, P, C = x.shape
    IN_F = IN_F_STATIC
    OUT_F = T - IN_F
    NW = 32
    n_rows = NW * ROWS_PER_W

    table = jnp.concatenate([W_obs, W_pred, W_person], axis=0)
    t_idx = jnp.arange(T, dtype=jnp.int32)
    obs_rows = jnp.asarray(in_F, jnp.int32) - 1 - t_idx
    pred_rows = W_obs.shape[0] + (t_idx - IN_F) + (
        jnp.asarray(out_F, jnp.int32) - OUT_F)
    time_rows = jnp.where(t_idx < IN_F, obs_rows, pred_rows)
    pers_rows = (W_obs.shape[0] + W_pred.shape[0]
                 + jnp.arange(P, dtype=jnp.int32)
                 + jnp.asarray(num_people, jnp.int32) - P)
    idx = jnp.concatenate([
        time_rows, pers_rows,
        jnp.zeros((n_rows - T - P,), jnp.int32),
    ])
    idx2d = jnp.pad(idx.reshape(NW, ROWS_PER_W), ((0, 0), (0, 8 - ROWS_PER_W)))

    emb3 = _sc_lookup_renorm(table, idx2d, n_rows)
    emb = emb3[:, :ROWS_PER_W, :].reshape(n_rows, C // 2)
    temb, pemb = emb[:T], emb[T:T + P]

    return pl.pallas_call(
        _tc_body,
        grid=(B // CHUNK,),
        in_specs=[
            pl.BlockSpec((CHUNK, T, P, C), lambda i: (i, 0, 0, 0)),
            pl.BlockSpec((T, C // 2), lambda i: (0, 0)),
            pl.BlockSpec((P, C // 2), lambda i: (0, 0)),
        ],
        out_specs=pl.BlockSpec((CHUNK, T, P, C), lambda i: (i, 0, 0, 0)),
        scratch_shapes=[pltpu.VMEM((T, P, C), jnp.float32)],
        out_shape=jax.ShapeDtypeStruct(x.shape, x.dtype),
    )(x, temb, pemb)


# speedup vs baseline: 1.1517x; 1.1517x over previous
import jax
import jax.numpy as jnp
from jax import lax
from jax.experimental import pallas as pl
from jax.experimental.pallas import tpu as pltpu

IN_F_STATIC = 50
CHUNK = 8


def _renorm(rows):
    norm = jnp.sqrt(jnp.sum(rows * rows, axis=-1, keepdims=True))
    scale = jnp.where(norm > 1.0, 1.0 / (norm + 1e-7), 1.0)
    return rows * scale


def _body(starts_ref, x_ref, wobs_ref, wpred_ref, wpers_ref, o_ref, bias_ref):
    T, P, C = x_ref.shape[1], x_ref.shape[2], x_ref.shape[3]
    H = C // 2
    IN_F = IN_F_STATIC
    OUT_F = T - IN_F

    @pl.when(pl.program_id(0) == 0)
    def _build_bias():
        obs = _renorm(wobs_ref[pl.ds(starts_ref[0], IN_F), :])
        pred = _renorm(wpred_ref[pl.ds(starts_ref[1], OUT_F), :])
        pers = _renorm(wpers_ref[pl.ds(starts_ref[2], P), :])

        fi = lax.broadcasted_iota(jnp.int32, (IN_F, IN_F), 0)
        fj = lax.broadcasted_iota(jnp.int32, (IN_F, IN_F), 1)
        flip = (fj == (IN_F - 1 - fi)).astype(jnp.float32)
        obs_r = lax.dot(flip, obs, precision=lax.Precision.HIGHEST)

        hr = lax.broadcasted_iota(jnp.int32, (H, C), 0)
        hc = lax.broadcasted_iota(jnp.int32, (H, C), 1)
        even = (hc == 2 * hr).astype(jnp.float32)
        odd = (hc == 2 * hr + 1).astype(jnp.float32)
        obs_part = lax.dot(obs_r, even, precision=lax.Precision.HIGHEST)
        pred_part = lax.dot(pred, even, precision=lax.Precision.HIGHEST)
        pers_part = lax.dot(pers, odd, precision=lax.Precision.HIGHEST)

        pers_b = pers_part[None, :, :]
        bias_ref[0:IN_F] = obs_part[:, None, :] + pers_b
        bias_ref[IN_F:T] = pred_part[:, None, :] + pers_b

    o_ref[...] = x_ref[...] + bias_ref[...]


def kernel(x, W_obs, W_pred, W_person, in_F, out_F, num_people):
    B, T, P, C = x.shape
    IN_F = IN_F_STATIC
    OUT_F = T - IN_F
    starts = jnp.stack([
        jnp.asarray(in_F, jnp.int32) - IN_F,
        jnp.asarray(out_F, jnp.int32) - OUT_F,
        jnp.asarray(num_people, jnp.int32) - P,
    ])

    grid = (B // CHUNK,)
    return pl.pallas_call(
        _body,
        grid_spec=pltpu.PrefetchScalarGridSpec(
            num_scalar_prefetch=1,
            grid=grid,
            in_specs=[
                pl.BlockSpec((CHUNK, T, P, C), lambda i, s: (i, 0, 0, 0)),
                pl.BlockSpec(W_obs.shape, lambda i, s: (0, 0)),
                pl.BlockSpec(W_pred.shape, lambda i, s: (0, 0)),
                pl.BlockSpec(W_person.shape, lambda i, s: (0, 0)),
            ],
            out_specs=pl.BlockSpec((CHUNK, T, P, C), lambda i, s: (i, 0, 0, 0)),
            scratch_shapes=[pltpu.VMEM((T, P, C), jnp.float32)],
        ),
        out_shape=jax.ShapeDtypeStruct(x.shape, x.dtype),
        compiler_params=pltpu.CompilerParams(vmem_limit_bytes=128 * 1024 * 1024),
    )(starts, x, W_obs, W_pred, W_person)
